# baseline (speedup 1.0000x reference)
import functools
import jax
import jax.numpy as jnp
from jax import lax
from jax.experimental import pallas as pl
from jax.experimental.pallas import tpu as pltpu
from jax.experimental.pallas import tpu_sc as plsc

_S_SC = 16
_R = 224
_IDX = 112


def _make_sc_kernel(N, S_sc, s_off, HWD, E, dtype):
    info = plsc.get_sparse_core_info()
    NC, NS = info.num_cores, info.num_subcores
    NW = NC * NS
    n_slabs = N * S_sc
    assert n_slabs % NW == 0
    spw = n_slabs // NW
    n_chunks = HWD // _R
    if spw % N == 0:
        n_bufs = spw // N
    else:
        assert N % spw == 0
        n_bufs = 1
    mesh = plsc.VectorSubcoreMesh(core_axis_name="c", subcore_axis_name="s")

    @functools.partial(
        pl.kernel,
        mesh=mesh,
        out_type=jax.ShapeDtypeStruct((N, S_sc, HWD, E), dtype),
        scratch_types=(
            [pltpu.VMEM((_R, E), dtype) for _ in range(n_bufs)]
            + [pltpu.VMEM((_IDX,), jnp.int32) for _ in range(n_bufs)]
            + [pltpu.SemaphoreType.DMA((n_bufs,)),
               pltpu.SemaphoreType.DMA((n_bufs,))]
        ),
    )
    def sc_kernel(table_hbm, out_hbm, *scratch):
        bufs = scratch[:n_bufs]
        idxs = scratch[n_bufs:2 * n_bufs]
        gsem, ssem = scratch[2 * n_bufs:]
        wid = lax.axis_index("s") * NC + lax.axis_index("c")
        gathers = [[] for _ in range(n_bufs)]
        scatters = [[] for _ in range(n_bufs)]
        svals = []
        for b in range(n_bufs):
            if spw % N == 0:
                s_local = wid * n_bufs + b
            else:
                s_local = wid // (N // spw)
            svals.append(s_local)
            splat = jnp.full((16,), s_local + s_off, jnp.int32)
            for i in range(_IDX // 16):
                idxs[b][pl.ds(i * 16, 16)] = splat
            for h in range(_R // _IDX):
                cp = pltpu.make_async_copy(
                    table_hbm.at[idxs[b]],
                    bufs[b].at[pl.ds(h * _IDX, _IDX)],
                    gsem.at[b],
                )
                cp.start()
                gathers[b].append(cp)
        for b in range(n_bufs):
            for cp in gathers[b]:
                cp.wait()
            if spw % N == 0:
                ns = list(range(N))
            else:
                n0 = (wid % (N // spw)) * spw
                ns = [n0 + k for k in range(spw)]
            for n in ns:
                for j in range(n_chunks):
                    cp = pltpu.make_async_copy(
                        bufs[b],
                        out_hbm.at[n, svals[b], pl.ds(j * _R, _R)],
                        ssem.at[b],
                    )
                    cp.start()
                    scatters[b].append(cp)
        for b in range(n_bufs):
            for cp in scatters[b]:
                cp.wait()

    return sc_kernel


def _tc_bcast_kernel(table_ref, out_ref):
    row = table_ref[0, 0, :]
    out_ref[...] = jnp.broadcast_to(row[None, None, None, :], out_ref.shape)


def _tc_part(table3, N, S_tc, HWD, E):
    return pl.pallas_call(
        _tc_bcast_kernel,
        grid=(S_tc,),
        in_specs=[pl.BlockSpec((1, 1, E), lambda s: (s, 0, 0))],
        out_specs=pl.BlockSpec((N, 1, HWD, E), lambda s: (0, s, 0, 0)),
        out_shape=jax.ShapeDtypeStruct((N, S_tc, HWD, E), table3.dtype),
    )(table3)


def kernel(x, table):
    N, S, H, W, D = x.shape
    T, E = table.shape
    HWD = H * W * D
    S_tc = S - _S_SC

    sc_out = _make_sc_kernel(N, _S_SC, S_tc, HWD, E, table.dtype)(table)
    tc_out = _tc_part(table.reshape(T, 1, E), N, S_tc, HWD, E)
    out = jnp.concatenate([tc_out, sc_out], axis=1)
    return out.reshape(N, S, H, W, D, E)


# speedup vs baseline: 1.9273x; 1.9273x over previous
import functools
import jax
import jax.numpy as jnp
from jax import lax
from jax.experimental import pallas as pl
from jax.experimental.pallas import tpu as pltpu
from jax.experimental.pallas import tpu_sc as plsc

_S_SC = 8
_R = 224
_IDX = 112
_CH = 8


def _make_sc_kernel(N, S, s_off, HWD, E, dtype):
    info = plsc.get_sparse_core_info()
    NC, NS = info.num_cores, info.num_subcores
    NW = NC * NS
    S_sc = S - s_off
    n_slabs = N * S_sc
    assert n_slabs % NW == 0
    spw = n_slabs // NW
    n_chunks = HWD // _R
    if spw % N == 0:
        n_bufs = spw // N
    else:
        assert N % spw == 0
        n_bufs = 1
    mesh = plsc.VectorSubcoreMesh(core_axis_name="c", subcore_axis_name="s")

    @functools.partial(
        pl.kernel,
        mesh=mesh,
        out_type=jax.ShapeDtypeStruct((N, S, HWD, E), dtype),
        scratch_types=(
            [pltpu.VMEM((_R, E), dtype) for _ in range(n_bufs)]
            + [pltpu.VMEM((_IDX,), jnp.int32) for _ in range(n_bufs)]
            + [pltpu.SemaphoreType.DMA((n_bufs,)),
               pltpu.SemaphoreType.DMA((n_bufs,))]
        ),
    )
    def sc_kernel(table_hbm, out_hbm, *scratch):
        bufs = scratch[:n_bufs]
        idxs = scratch[n_bufs:2 * n_bufs]
        gsem, ssem = scratch[2 * n_bufs:]
        wid = lax.axis_index("s") * NC + lax.axis_index("c")
        gathers = [[] for _ in range(n_bufs)]
        scatters = [[] for _ in range(n_bufs)]
        svals = []
        for b in range(n_bufs):
            if spw % N == 0:
                s = wid * n_bufs + b + s_off
            else:
                s = wid // (N // spw) + s_off
            svals.append(s)
            splat = jnp.full((16,), s, jnp.int32)
            for i in range(_IDX // 16):
                idxs[b][pl.ds(i * 16, 16)] = splat
            for h in range(_R // _IDX):
                cp = pltpu.make_async_copy(
                    table_hbm.at[idxs[b]],
                    bufs[b].at[pl.ds(h * _IDX, _IDX)],
                    gsem.at[b],
                )
                cp.start()
                gathers[b].append(cp)
        for b in range(n_bufs):
            for cp in gathers[b]:
                cp.wait()
            if spw % N == 0:
                ns = list(range(N))
            else:
                n0 = (wid % (N // spw)) * spw
                ns = [n0 + k for k in range(spw)]
            for n in ns:
                for j in range(n_chunks):
                    cp = pltpu.make_async_copy(
                        bufs[b],
                        out_hbm.at[n, svals[b], pl.ds(j * _R, _R)],
                        ssem.at[b],
                    )
                    cp.start()
                    scatters[b].append(cp)
        for b in range(n_bufs):
            for cp in scatters[b]:
                cp.wait()

    return sc_kernel


def _tc_fill_kernel(S_tc, table_ref, partial_ref, out_ref, buf0, buf1, sem):
    del partial_ref
    N, S, HWD, E = out_ref.shape
    G = S_tc // _CH
    bufs = (buf0, buf1)

    def copies(g):
        buf = bufs[g % 2]
        return [
            pltpu.make_async_copy(
                buf, out_ref.at[n, pl.ds(g * _CH, _CH)], sem.at[g % 2]
            )
            for n in range(N)
        ]

    for g in range(G):
        if g >= 2:
            for c in copies(g - 2):
                c.wait()
        rows = table_ref[pl.ds(g * _CH, _CH), :]
        bufs[g % 2][...] = jnp.broadcast_to(rows[:, None, :], (_CH, HWD, E))
        for c in copies(g):
            c.start()
    for g in range(max(G - 2, 0), G):
        for c in copies(g):
            c.wait()


def kernel(x, table):
    N, S, H, W, D = x.shape
    T, E = table.shape
    HWD = H * W * D
    S_tc = S - _S_SC

    partial = _make_sc_kernel(N, S, S_tc, HWD, E, table.dtype)(table)
    out = pl.pallas_call(
        functools.partial(_tc_fill_kernel, S_tc),
        in_specs=[
            pl.BlockSpec(memory_space=pltpu.VMEM),
            pl.BlockSpec(memory_space=pl.ANY),
        ],
        out_specs=pl.BlockSpec(memory_space=pl.ANY),
        out_shape=jax.ShapeDtypeStruct((N, S, HWD, E), table.dtype),
        scratch_shapes=[
            pltpu.VMEM((_CH, HWD, E), table.dtype),
            pltpu.VMEM((_CH, HWD, E), table.dtype),
            pltpu.SemaphoreType.DMA((2,)),
        ],
        input_output_aliases={1: 0},
    )(table, partial)
    return out.reshape(N, S, H, W, D, E)


# speedup vs baseline: 3.7954x; 1.9693x over previous
import jax
import jax.numpy as jnp
from jax.experimental import pallas as pl
from jax.experimental.pallas import tpu as pltpu


_CH = 32


def _fill_and_copy_kernel(table_ref, out_ref, buf0, buf1, sem):
    N, S, HWD, E = out_ref.shape
    G = S // _CH
    bufs = (buf0, buf1)

    def copies(g):
        buf = bufs[g % 2]
        return [
            pltpu.make_async_copy(
                buf, out_ref.at[n, pl.ds(g * _CH, _CH)], sem.at[g % 2]
            )
            for n in range(N)
        ]

    for g in range(G):
        if g >= 2:
            for c in copies(g - 2):
                c.wait()
        rows = table_ref[pl.ds(g * _CH, _CH), :]
        bufs[g % 2][...] = jnp.broadcast_to(rows[:, None, :], (_CH, HWD, E))
        for c in copies(g):
            c.start()
    for g in range(max(G - 2, 0), G):
        for c in copies(g):
            c.wait()


def kernel(x, table):
    N, S, H, W, D = x.shape
    T, E = table.shape
    HWD = H * W * D

    out = pl.pallas_call(
        _fill_and_copy_kernel,
        in_specs=[pl.BlockSpec(memory_space=pltpu.VMEM)],
        out_specs=pl.BlockSpec(memory_space=pl.ANY),
        out_shape=jax.ShapeDtypeStruct((N, S, HWD, E), table.dtype),
        scratch_shapes=[
            pltpu.VMEM((_CH, HWD, E), table.dtype),
            pltpu.VMEM((_CH, HWD, E), table.dtype),
            pltpu.SemaphoreType.DMA((2,)),
        ],
    )(table)
    return out.reshape(N, S, H, W, D, E)


# speedup vs baseline: 3.8743x; 1.0208x over previous
import jax
import jax.numpy as jnp
from jax.experimental import pallas as pl
from jax.experimental.pallas import tpu as pltpu


_CH = 8


def _fill_and_copy_kernel(table_ref, out_ref, buf0, buf1, sem):
    N, S, HWD, E = out_ref.shape
    G = S // _CH
    bufs = (buf0, buf1)

    def copies(g):
        buf = bufs[g % 2]
        return [
            pltpu.make_async_copy(
                buf, out_ref.at[n, pl.ds(g * _CH, _CH)], sem.at[g % 2]
            )
            for n in range(N)
        ]

    for g in range(G):
        if g >= 2:
            for c in copies(g - 2):
                c.wait()
        rows = table_ref[pl.ds(g * _CH, _CH), :]
        bufs[g % 2][...] = jnp.broadcast_to(rows[:, None, :], (_CH, HWD, E))
        for c in copies(g):
            c.start()
    for g in range(max(G - 2, 0), G):
        for c in copies(g):
            c.wait()


def kernel(x, table):
    N, S, H, W, D = x.shape
    T, E = table.shape
    HWD = H * W * D

    out = pl.pallas_call(
        _fill_and_copy_kernel,
        in_specs=[pl.BlockSpec(memory_space=pltpu.VMEM)],
        out_specs=pl.BlockSpec(memory_space=pl.ANY),
        out_shape=jax.ShapeDtypeStruct((N, S, HWD, E), table.dtype),
        scratch_shapes=[
            pltpu.VMEM((_CH, HWD, E), table.dtype),
            pltpu.VMEM((_CH, HWD, E), table.dtype),
            pltpu.SemaphoreType.DMA((2,)),
        ],
    )(table)
    return out.reshape(N, S, H, W, D, E)
